# baseline (speedup 1.0000x reference)
import functools

import jax
import jax.numpy as jnp
from jax import lax
from jax.experimental import pallas as pl
from jax.experimental.pallas import tpu as pltpu
from jax.experimental.pallas import tpu_sc as plsc

_L = 16


def _make_sc_kernel(B, SEQ, E, V):
    info = plsc.get_sparse_core_info()
    NC, NS = info.num_cores, info.num_subcores
    NW = NC * NS
    rows_total = B * SEQ
    rows_per_w = rows_total // NW
    CHUNK = 128
    n_chunks = rows_per_w // CHUNK
    assert rows_per_w % CHUNK == 0
    groups = E // _L

    NBUF = 4
    assert n_chunks % NBUF == 0

    mesh = plsc.VectorSubcoreMesh(core_axis_name="c", subcore_axis_name="s")

    @functools.partial(
        pl.kernel,
        out_type=jax.ShapeDtypeStruct((rows_total, E), jnp.float32),
        mesh=mesh,
        scratch_types=[
            pltpu.VMEM((n_chunks, CHUNK), jnp.int32),
            pltpu.VMEM((SEQ, E), jnp.float32),
            pltpu.VMEM((NBUF, CHUNK, E), jnp.float32),
            pltpu.SemaphoreType.DMA((NBUF,)),
            pltpu.SemaphoreType.DMA((NBUF,)),
        ],
    )
    def k(idx_hbm, text_hbm, pos_hbm, out_hbm, idx_v, pos_v, rows_v, gsem, osem):
        wid = lax.axis_index("s") * NC + lax.axis_index("c")
        base_row = wid * rows_per_w
        pltpu.sync_copy(idx_hbm.at[wid], idx_v)
        pltpu.sync_copy(pos_hbm, pos_v)

        def gdesc(c, b):
            return pltpu.make_async_copy(
                text_hbm.at[idx_v.at[c]], rows_v.at[b], gsem.at[b]
            )

        def odesc(c, b):
            return pltpu.make_async_copy(
                rows_v.at[b],
                out_hbm.at[pl.ds(base_row + c * CHUNK, CHUNK)],
                osem.at[b],
            )

        for b in range(NBUF - 1):
            gdesc(b, b).start()

        def group(g, carry):
            for b in range(NBUF):
                c = g * NBUF + b
                pb = (b + NBUF - 1) % NBUF

                @pl.when(c + NBUF - 1 < n_chunks)
                def _():
                    @pl.when(c >= 1)
                    def _():
                        odesc(c - 1, pb).wait()

                    gdesc(c + NBUF - 1, pb).start()

                gdesc(c, b).wait()
                phase = lax.rem(c * CHUNK, SEQ)
                UNROLL = 8

                @plsc.parallel_loop(0, CHUNK, step=UNROLL, carry=jnp.int32(0))
                def _add(r0, carry2, _b=b, _phase=phase):
                    for u in range(UNROLL):
                        r = r0 + u
                        l = lax.rem(_phase + r, SEQ)
                        for gi in range(groups):
                            sl = pl.ds(gi * _L, _L)
                            plsc.addupdate(
                                rows_v.at[_b, r, sl], pos_v[l, sl]
                            )
                    return carry2
                odesc(c, b).start()
            return carry

        lax.fori_loop(0, n_chunks // NBUF, group, 0)

        for b in range(NBUF):
            odesc(n_chunks - NBUF + b, b).wait()

    return k, NW, rows_per_w, CHUNK, n_chunks


def kernel(inputs, text_table, pos_table):
    B, SEQ = inputs.shape
    V, E = text_table.shape
    k, NW, rows_per_w, CHUNK, n_chunks = _make_sc_kernel(B, SEQ, E, V)
    idx = inputs.astype(jnp.int32).reshape(NW, n_chunks, CHUNK)
    out = k(idx, text_table, pos_table)
    return out.reshape(B, SEQ, E)


# speedup vs baseline: 1.1162x; 1.1162x over previous
import functools

import jax
import jax.numpy as jnp
from jax import lax
from jax.experimental import pallas as pl
from jax.experimental.pallas import tpu as pltpu
from jax.experimental.pallas import tpu_sc as plsc

_L = 16


def _make_sc_kernel(B, SEQ, E, V):
    info = plsc.get_sparse_core_info()
    NC, NS = info.num_cores, info.num_subcores
    NW = NC * NS
    rows_total = B * SEQ
    rows_per_w = rows_total // NW
    CHUNK = 128
    n_chunks = rows_per_w // CHUNK
    assert rows_per_w % CHUNK == 0
    groups = E // _L

    NBUF = 4
    assert n_chunks % NBUF == 0

    mesh = plsc.VectorSubcoreMesh(core_axis_name="c", subcore_axis_name="s")

    @functools.partial(
        pl.kernel,
        out_type=jax.ShapeDtypeStruct((rows_total, E), jnp.float32),
        mesh=mesh,
        scratch_types=[
            pltpu.VMEM((n_chunks, CHUNK), jnp.int32),
            pltpu.VMEM((SEQ, E), jnp.float32),
            pltpu.VMEM((NBUF, CHUNK, E), jnp.float32),
            pltpu.SemaphoreType.DMA((NBUF,)),
            pltpu.SemaphoreType.DMA((NBUF,)),
        ],
    )
    def k(idx_hbm, text_hbm, pos_hbm, out_hbm, idx_v, pos_v, rows_v, gsem, osem):
        wid = lax.axis_index("s") * NC + lax.axis_index("c")
        base_row = wid * rows_per_w
        pltpu.sync_copy(idx_hbm.at[wid], idx_v)
        pltpu.sync_copy(pos_hbm, pos_v)

        def gdesc(c, b):
            return pltpu.make_async_copy(
                text_hbm.at[idx_v.at[c]], rows_v.at[b], gsem.at[b]
            )

        def odesc(c, b):
            return pltpu.make_async_copy(
                rows_v.at[b],
                out_hbm.at[pl.ds(base_row + c * CHUNK, CHUNK)],
                osem.at[b],
            )

        for b in range(NBUF - 1):
            gdesc(b, b).start()

        def group(g, carry):
            for b in range(NBUF):
                c = g * NBUF + b
                pb = (b + NBUF - 1) % NBUF

                @pl.when(c + NBUF - 1 < n_chunks)
                def _():
                    @pl.when(c >= 1)
                    def _():
                        odesc(c - 1, pb).wait()

                    gdesc(c + NBUF - 1, pb).start()

                gdesc(c, b).wait()
                phase = lax.rem(c * CHUNK, SEQ)
                UNROLL = 2

                @plsc.parallel_loop(0, CHUNK, step=UNROLL, carry=jnp.int32(0))
                def _add(r0, carry2, _b=b, _phase=phase):
                    for u in range(UNROLL):
                        r = r0 + u
                        l = lax.rem(_phase + r, SEQ)
                        for gi in range(groups):
                            sl = pl.ds(gi * _L, _L)
                            plsc.addupdate(
                                rows_v.at[_b, r, sl], pos_v[l, sl]
                            )
                    return carry2
                odesc(c, b).start()
            return carry

        lax.fori_loop(0, n_chunks // NBUF, group, 0)

        for b in range(NBUF):
            odesc(n_chunks - NBUF + b, b).wait()

    return k, NW, rows_per_w, CHUNK, n_chunks


def kernel(inputs, text_table, pos_table):
    B, SEQ = inputs.shape
    V, E = text_table.shape
    k, NW, rows_per_w, CHUNK, n_chunks = _make_sc_kernel(B, SEQ, E, V)
    idx = inputs.astype(jnp.int32).reshape(NW, n_chunks, CHUNK)
    out = k(idx, text_table, pos_table)
    return out.reshape(B, SEQ, E)


# speedup vs baseline: 1.1296x; 1.0120x over previous
import functools

import jax
import jax.numpy as jnp
from jax import lax
from jax.experimental import pallas as pl
from jax.experimental.pallas import tpu as pltpu
from jax.experimental.pallas import tpu_sc as plsc

_L = 16


def _make_sc_kernel(B, SEQ, E, V):
    info = plsc.get_sparse_core_info()
    NC, NS = info.num_cores, info.num_subcores
    NW = NC * NS
    rows_total = B * SEQ
    rows_per_w = rows_total // NW
    CHUNK = 128
    n_chunks = rows_per_w // CHUNK
    assert rows_per_w % CHUNK == 0
    groups = E // _L

    NBUF = 4
    assert n_chunks % NBUF == 0

    mesh = plsc.VectorSubcoreMesh(core_axis_name="c", subcore_axis_name="s")

    @functools.partial(
        pl.kernel,
        out_type=jax.ShapeDtypeStruct((rows_total, E), jnp.float32),
        mesh=mesh,
        scratch_types=[
            pltpu.VMEM((n_chunks, CHUNK), jnp.int32),
            pltpu.VMEM((SEQ, E), jnp.float32),
            pltpu.VMEM((NBUF, CHUNK, E), jnp.float32),
            pltpu.SemaphoreType.DMA((NBUF,)),
            pltpu.SemaphoreType.DMA((NBUF,)),
        ],
    )
    def k(idx_hbm, text_hbm, pos_hbm, out_hbm, idx_v, pos_v, rows_v, gsem, osem):
        wid = lax.axis_index("s") * NC + lax.axis_index("c")
        base_row = wid * rows_per_w
        pltpu.sync_copy(idx_hbm.at[wid], idx_v)
        pltpu.sync_copy(pos_hbm, pos_v)

        def gdesc(c, b):
            return pltpu.make_async_copy(
                text_hbm.at[idx_v.at[c]], rows_v.at[b], gsem.at[b]
            )

        def odesc(c, b):
            return pltpu.make_async_copy(
                rows_v.at[b],
                out_hbm.at[pl.ds(base_row + c * CHUNK, CHUNK)],
                osem.at[b],
            )

        for b in range(NBUF - 1):
            gdesc(b, b).start()

        def group(g, carry):
            for b in range(NBUF):
                c = g * NBUF + b
                pb = (b + NBUF - 1) % NBUF

                @pl.when(c + NBUF - 1 < n_chunks)
                def _():
                    @pl.when(c >= 1)
                    def _():
                        odesc(c - 1, pb).wait()

                    gdesc(c + NBUF - 1, pb).start()

                gdesc(c, b).wait()
                phase = lax.rem(c * CHUNK, SEQ)
                UNROLL = 1

                @plsc.parallel_loop(0, CHUNK, step=UNROLL, carry=jnp.int32(0))
                def _add(r0, carry2, _b=b, _phase=phase):
                    for u in range(UNROLL):
                        r = r0 + u
                        l = lax.rem(_phase + r, SEQ)
                        for gi in range(groups):
                            sl = pl.ds(gi * _L, _L)
                            plsc.addupdate(
                                rows_v.at[_b, r, sl], pos_v[l, sl]
                            )
                    return carry2
                odesc(c, b).start()
            return carry

        lax.fori_loop(0, n_chunks // NBUF, group, 0)

        for b in range(NBUF):
            odesc(n_chunks - NBUF + b, b).wait()

    return k, NW, rows_per_w, CHUNK, n_chunks


def kernel(inputs, text_table, pos_table):
    B, SEQ = inputs.shape
    V, E = text_table.shape
    k, NW, rows_per_w, CHUNK, n_chunks = _make_sc_kernel(B, SEQ, E, V)
    idx = inputs.astype(jnp.int32).reshape(NW, n_chunks, CHUNK)
    out = k(idx, text_table, pos_table)
    return out.reshape(B, SEQ, E)


# speedup vs baseline: 1.3921x; 1.2324x over previous
import functools

import jax
import jax.numpy as jnp
from jax import lax
from jax.experimental import pallas as pl
from jax.experimental.pallas import tpu as pltpu
from jax.experimental.pallas import tpu_sc as plsc


def _make_sc_kernel(B, SEQ, E, V):
    info = plsc.get_sparse_core_info()
    NC, NS = info.num_cores, info.num_subcores
    NW = NC * NS
    rows_total = B * SEQ
    rows_per_w = rows_total // NW
    CHUNK = 128
    n_chunks = rows_per_w // CHUNK
    assert rows_per_w % CHUNK == 0

    NBUF = 4
    assert n_chunks % NBUF == 0

    mesh = plsc.VectorSubcoreMesh(core_axis_name="c", subcore_axis_name="s")

    @functools.partial(
        pl.kernel,
        out_type=jax.ShapeDtypeStruct((rows_total, E), jnp.float32),
        mesh=mesh,
        scratch_types=[
            pltpu.VMEM((n_chunks, CHUNK), jnp.int32),
            pltpu.VMEM((NBUF, CHUNK, E), jnp.float32),
            pltpu.VMEM_SHARED((2 * SEQ, E), jnp.float32),
            pltpu.SemaphoreType.DMA((NBUF,)),
            pltpu.SemaphoreType.DMA((NBUF,)),
            pltpu.SemaphoreType.DMA((NBUF,)),
        ],
    )
    def k(idx_hbm, text_hbm, pos2_hbm, out_hbm, idx_v, rows_v, pos_s,
          psem, gsem, osem):
        cid = lax.axis_index("c")
        sid = lax.axis_index("s")
        wid = sid * NC + cid
        base_row = wid * rows_per_w
        pltpu.sync_copy(idx_hbm.at[wid], idx_v)

        @pl.when(sid == 0)
        def _():
            pltpu.sync_copy(pos2_hbm, pos_s)

        plsc.subcore_barrier()

        def pdesc(c, b):
            phase = lax.rem(c * CHUNK, SEQ)
            return pltpu.make_async_copy(
                pos_s.at[pl.ds(phase, CHUNK)], rows_v.at[b], psem.at[b]
            )

        def gdesc(c, b):
            return pltpu.make_async_copy(
                text_hbm.at[idx_v.at[c]], rows_v.at[b], gsem.at[b]
            )

        def odesc(c, b):
            return pltpu.make_async_copy(
                rows_v.at[b],
                out_hbm.at[pl.ds(base_row + c * CHUNK, CHUNK)],
                osem.at[b],
            )

        for c0 in range(NBUF - 1):
            pdesc(c0, c0).start()
        for c0 in range(NBUF - 2):
            pdesc(c0, c0).wait()
            gdesc(c0, c0).start(add=True)

        def group(g, carry):
            for b in range(NBUF):
                c = g * NBUF + b
                pb = (b + NBUF - 1) % NBUF
                qb = (b + NBUF - 2) % NBUF

                @pl.when(c + NBUF - 1 < n_chunks)
                def _():
                    @pl.when(c >= 1)
                    def _():
                        odesc(c - 1, pb).wait()

                    pdesc(c + NBUF - 1, pb).start()

                @pl.when(c + NBUF - 2 < n_chunks)
                def _():
                    pdesc(c + NBUF - 2, qb).wait()
                    gdesc(c + NBUF - 2, qb).start(add=True)

                gdesc(c, b).wait()
                odesc(c, b).start()
            return carry

        lax.fori_loop(0, n_chunks // NBUF, group, 0)

        for b in range(NBUF):
            odesc(n_chunks - NBUF + b, b).wait()

    return k, NW, rows_per_w, CHUNK, n_chunks


def kernel(inputs, text_table, pos_table):
    B, SEQ = inputs.shape
    V, E = text_table.shape
    k, NW, rows_per_w, CHUNK, n_chunks = _make_sc_kernel(B, SEQ, E, V)
    idx = inputs.astype(jnp.int32).reshape(NW, n_chunks, CHUNK)
    pos2 = jnp.concatenate([pos_table, pos_table], axis=0)
    out = k(idx, text_table, pos2)
    return out.reshape(B, SEQ, E)


# speedup vs baseline: 1.3926x; 1.0004x over previous
import functools

import jax
import jax.numpy as jnp
from jax import lax
from jax.experimental import pallas as pl
from jax.experimental.pallas import tpu as pltpu
from jax.experimental.pallas import tpu_sc as plsc


def _make_sc_kernel(B, SEQ, E, V):
    info = plsc.get_sparse_core_info()
    NC, NS = info.num_cores, info.num_subcores
    NW = NC * NS
    rows_total = B * SEQ
    rows_per_w = rows_total // NW
    CHUNK = 128
    n_chunks = rows_per_w // CHUNK
    assert rows_per_w % CHUNK == 0

    NBUF = 5
    assert n_chunks % NBUF == 0

    mesh = plsc.VectorSubcoreMesh(core_axis_name="c", subcore_axis_name="s")

    @functools.partial(
        pl.kernel,
        out_type=jax.ShapeDtypeStruct((rows_total, E), jnp.float32),
        mesh=mesh,
        scratch_types=[
            pltpu.VMEM((n_chunks, CHUNK), jnp.int32),
            pltpu.VMEM((NBUF, CHUNK, E), jnp.float32),
            pltpu.VMEM_SHARED((2 * SEQ, E), jnp.float32),
            pltpu.SemaphoreType.DMA((NBUF,)),
            pltpu.SemaphoreType.DMA((NBUF,)),
            pltpu.SemaphoreType.DMA((NBUF,)),
        ],
    )
    def k(idx_hbm, text_hbm, pos2_hbm, out_hbm, idx_v, rows_v, pos_s,
          psem, gsem, osem):
        cid = lax.axis_index("c")
        sid = lax.axis_index("s")
        wid = sid * NC + cid
        base_row = wid * rows_per_w
        pltpu.sync_copy(idx_hbm.at[wid], idx_v)

        @pl.when(sid == 0)
        def _():
            pltpu.sync_copy(pos2_hbm, pos_s)

        plsc.subcore_barrier()

        def pdesc(c, b):
            phase = lax.rem(c * CHUNK, SEQ)
            return pltpu.make_async_copy(
                pos_s.at[pl.ds(phase, CHUNK)], rows_v.at[b], psem.at[b]
            )

        def gdesc(c, b):
            return pltpu.make_async_copy(
                text_hbm.at[idx_v.at[c]], rows_v.at[b], gsem.at[b]
            )

        def odesc(c, b):
            return pltpu.make_async_copy(
                rows_v.at[b],
                out_hbm.at[pl.ds(base_row + c * CHUNK, CHUNK)],
                osem.at[b],
            )

        for c0 in range(NBUF - 1):
            pdesc(c0, c0).start()
        for c0 in range(NBUF - 2):
            pdesc(c0, c0).wait()
            gdesc(c0, c0).start(add=True)

        def group(g, carry):
            for b in range(NBUF):
                c = g * NBUF + b
                pb = (b + NBUF - 1) % NBUF
                qb = (b + NBUF - 2) % NBUF

                @pl.when(c + NBUF - 1 < n_chunks)
                def _():
                    @pl.when(c >= 1)
                    def _():
                        odesc(c - 1, pb).wait()

                    pdesc(c + NBUF - 1, pb).start()

                @pl.when(c + NBUF - 2 < n_chunks)
                def _():
                    pdesc(c + NBUF - 2, qb).wait()
                    gdesc(c + NBUF - 2, qb).start(add=True)

                gdesc(c, b).wait()
                odesc(c, b).start()
            return carry

        lax.fori_loop(0, n_chunks // NBUF, group, 0)

        for b in range(NBUF):
            odesc(n_chunks - NBUF + b, b).wait()

    return k, NW, rows_per_w, CHUNK, n_chunks


def kernel(inputs, text_table, pos_table):
    B, SEQ = inputs.shape
    V, E = text_table.shape
    k, NW, rows_per_w, CHUNK, n_chunks = _make_sc_kernel(B, SEQ, E, V)
    idx = inputs.astype(jnp.int32).reshape(NW, n_chunks, CHUNK)
    pos2 = jnp.concatenate([pos_table, pos_table], axis=0)
    out = k(idx, text_table, pos2)
    return out.reshape(B, SEQ, E)
